# baseline (speedup 1.0000x reference)
import functools

import jax
import jax.numpy as jnp
from jax.experimental import pallas as pl
from jax.experimental.pallas import tpu as pltpu

_DIM = 64
_N_EMBED = 1024
_TOKENS = 16384
_BLK = 2048
_NUM_BLOCKS = _TOKENS // _BLK


def _vq_body(x_ref, e_ref, q_ref, ind_ref, diff_ref, ppl_ref, cnt_ref, dsum_ref):
    i = pl.program_id(0)

    @pl.when(i == 0)
    def _init():
        cnt_ref[...] = jnp.zeros_like(cnt_ref)
        dsum_ref[0, 0] = 0.0

    x = x_ref[...]
    e = e_ref[...]
    scores = jax.lax.dot_general(
        x, e, (((1,), (0,)), ((), ())), preferred_element_type=jnp.float32)
    x_sq = jnp.sum(x * x, axis=1, keepdims=True)
    e_sq = jnp.sum(e * e, axis=0, keepdims=True)
    neg_dist = -(x_sq - 2.0 * scores + e_sq)

    ind = jnp.argmax(neg_dist, axis=1).astype(jnp.int32)
    iota = jax.lax.broadcasted_iota(jnp.int32, (_BLK, _N_EMBED), 1)
    onehot = (iota == ind[:, None]).astype(jnp.float32)
    q = jax.lax.dot_general(
        onehot, e, (((1,), (1,)), ((), ())), preferred_element_type=jnp.float32)

    q_ref[...] = x + (q - x)
    ind_ref[...] = ind

    ones = jnp.ones((1, _BLK), dtype=jnp.float32)
    cnt_ref[...] += jax.lax.dot_general(
        ones, onehot, (((1,), (0,)), ((), ())),
        preferred_element_type=jnp.float32)[0]
    dsum_ref[0, 0] += jnp.sum((q - x) ** 2)

    @pl.when(i == _NUM_BLOCKS - 1)
    def _fin():
        diff_ref[...] = jnp.reshape(dsum_ref[0, 0] / float(_TOKENS * _DIM), (1, 1))
        p = cnt_ref[...] / float(_TOKENS)
        ent = jnp.sum(p * jnp.log(jnp.clip(p, 1e-7, None)), keepdims=True)
        ppl_ref[...] = jnp.exp(-ent).reshape(1, 1)


@functools.partial(jax.jit, static_argnames=())
def kernel(input, embed):
    flat = input.reshape(-1, _DIM)
    q, ind, diff, ppl = pl.pallas_call(
        _vq_body,
        grid=(_NUM_BLOCKS,),
        in_specs=[
            pl.BlockSpec((_BLK, _DIM), lambda i: (i, 0)),
            pl.BlockSpec((_DIM, _N_EMBED), lambda i: (0, 0)),
        ],
        out_specs=[
            pl.BlockSpec((_BLK, _DIM), lambda i: (i, 0)),
            pl.BlockSpec((_BLK,), lambda i: (i,)),
            pl.BlockSpec((1, 1), lambda i: (0, 0)),
            pl.BlockSpec((1, 1), lambda i: (0, 0)),
        ],
        out_shape=[
            jax.ShapeDtypeStruct((_TOKENS, _DIM), jnp.float32),
            jax.ShapeDtypeStruct((_TOKENS,), jnp.int32),
            jax.ShapeDtypeStruct((1, 1), jnp.float32),
            jax.ShapeDtypeStruct((1, 1), jnp.float32),
        ],
        scratch_shapes=[
            pltpu.VMEM((_N_EMBED,), jnp.float32),
            pltpu.SMEM((1, 1), jnp.float32),
        ],
    )(flat, embed)
    quantize_st = q.reshape(input.shape)
    embed_ind = ind.reshape(input.shape[:-1])
    return quantize_st, diff[0, 0], embed_ind, ppl[0, 0]


# speedup vs baseline: 1.0775x; 1.0775x over previous
import functools

import jax
import jax.numpy as jnp
from jax.experimental import pallas as pl
from jax.experimental.pallas import tpu as pltpu

_DIM = 64
_N_EMBED = 1024
_ROWS = 16
_COLS = 1024
_TOKENS = _ROWS * _COLS
_BR = 2
_BLK = _BR * _COLS
_NUM_BLOCKS = _ROWS // _BR


def _vq_body(x_ref, e_ref, q_ref, ind_ref, diff_ref, ppl_ref, cnt_ref, dsum_ref):
    i = pl.program_id(0)

    @pl.when(i == 0)
    def _init():
        cnt_ref[...] = jnp.zeros_like(cnt_ref)
        dsum_ref[0, 0] = 0.0

    x = x_ref[...].reshape(_BLK, _DIM)
    e = e_ref[...]
    scores = jax.lax.dot_general(
        x, e, (((1,), (0,)), ((), ())), preferred_element_type=jnp.float32)
    x_sq = jnp.sum(x * x, axis=1, keepdims=True)
    e_sq = jnp.sum(e * e, axis=0, keepdims=True)
    dist = x_sq - 2.0 * scores + e_sq

    ind = jnp.argmin(dist, axis=1).astype(jnp.int32)
    iota = jax.lax.broadcasted_iota(jnp.int32, (_BLK, _N_EMBED), 1)
    onehot = (iota == ind[:, None]).astype(jnp.float32)
    q = jax.lax.dot_general(
        onehot, e, (((1,), (1,)), ((), ())), preferred_element_type=jnp.float32)

    q_ref[...] = (x + (q - x)).reshape(_BR, _COLS, _DIM)
    ind_ref[...] = ind

    ones = jnp.ones((1, _BLK), dtype=jnp.float32)
    cnt_ref[...] += jax.lax.dot_general(
        ones, onehot, (((1,), (0,)), ((), ())),
        preferred_element_type=jnp.float32)[0]
    dsum_ref[0, 0] += jnp.sum((q - x) ** 2)

    @pl.when(i == _NUM_BLOCKS - 1)
    def _fin():
        diff_ref[...] = jnp.reshape(dsum_ref[0, 0] / float(_TOKENS * _DIM), (1, 1))
        p = cnt_ref[...] / float(_TOKENS)
        ent = jnp.sum(p * jnp.log(jnp.clip(p, 1e-7, None)), keepdims=True)
        ppl_ref[...] = jnp.exp(-ent).reshape(1, 1)


@functools.partial(jax.jit, static_argnames=())
def kernel(input, embed):
    q, ind, diff, ppl = pl.pallas_call(
        _vq_body,
        grid=(_NUM_BLOCKS,),
        in_specs=[
            pl.BlockSpec((_BR, _COLS, _DIM), lambda i: (i, 0, 0)),
            pl.BlockSpec((_DIM, _N_EMBED), lambda i: (0, 0)),
        ],
        out_specs=[
            pl.BlockSpec((_BR, _COLS, _DIM), lambda i: (i, 0, 0)),
            pl.BlockSpec((_BLK,), lambda i: (i,)),
            pl.BlockSpec((1, 1), lambda i: (0, 0)),
            pl.BlockSpec((1, 1), lambda i: (0, 0)),
        ],
        out_shape=[
            jax.ShapeDtypeStruct((_ROWS, _COLS, _DIM), jnp.float32),
            jax.ShapeDtypeStruct((_TOKENS,), jnp.int32),
            jax.ShapeDtypeStruct((1, 1), jnp.float32),
            jax.ShapeDtypeStruct((1, 1), jnp.float32),
        ],
        scratch_shapes=[
            pltpu.VMEM((_N_EMBED,), jnp.float32),
            pltpu.SMEM((1, 1), jnp.float32),
        ],
    )(input, embed)
    return q, diff[0, 0], ind.reshape(_ROWS, _COLS), ppl[0, 0]


# speedup vs baseline: 1.1039x; 1.0245x over previous
import functools

import jax
import jax.numpy as jnp
from jax.experimental import pallas as pl
from jax.experimental.pallas import tpu as pltpu

_DIM = 64
_N_EMBED = 1024
_ROWS = 16
_COLS = 1024
_TOKENS = _ROWS * _COLS
_BR = 4
_BLK = _BR * _COLS
_NUM_BLOCKS = _ROWS // _BR


def _vq_body(x_ref, e_ref, q_ref, ind_ref, diff_ref, ppl_ref, cnt_ref, dsum_ref):
    i = pl.program_id(0)

    @pl.when(i == 0)
    def _init():
        cnt_ref[...] = jnp.zeros_like(cnt_ref)
        dsum_ref[0, 0] = 0.0

    x = x_ref[...].reshape(_BLK, _DIM)
    e = e_ref[...]
    neg2_scores = jax.lax.dot_general(
        x * (-2.0), e, (((1,), (0,)), ((), ())),
        preferred_element_type=jnp.float32)
    x_sq = jnp.sum(x * x, axis=1, keepdims=True)
    e_sq = jnp.sum(e * e, axis=0, keepdims=True)
    dist = (x_sq + neg2_scores) + e_sq

    ind = jnp.argmin(dist, axis=1).astype(jnp.int32)
    iota = jax.lax.broadcasted_iota(jnp.int32, (_BLK, _N_EMBED), 1)
    onehot = (iota == ind[:, None]).astype(jnp.float32)
    q = jax.lax.dot_general(
        onehot, e, (((1,), (1,)), ((), ())), preferred_element_type=jnp.float32)

    q_ref[...] = (x + (q - x)).reshape(_BR, _COLS, _DIM)
    ind_ref[...] = ind

    ones = jnp.ones((1, _BLK), dtype=jnp.float32)
    cnt_ref[...] += jax.lax.dot_general(
        ones, onehot, (((1,), (0,)), ((), ())),
        preferred_element_type=jnp.float32)[0]
    dsum_ref[0, 0] += jnp.sum((q - x) ** 2)

    @pl.when(i == _NUM_BLOCKS - 1)
    def _fin():
        diff_ref[...] = jnp.reshape(dsum_ref[0, 0] / float(_TOKENS * _DIM), (1, 1))
        p = cnt_ref[...] / float(_TOKENS)
        ent = jnp.sum(p * jnp.log(jnp.clip(p, 1e-7, None)), keepdims=True)
        ppl_ref[...] = jnp.exp(-ent).reshape(1, 1)


@functools.partial(jax.jit, static_argnames=())
def kernel(input, embed):
    q, ind, diff, ppl = pl.pallas_call(
        _vq_body,
        grid=(_NUM_BLOCKS,),
        in_specs=[
            pl.BlockSpec((_BR, _COLS, _DIM), lambda i: (i, 0, 0)),
            pl.BlockSpec((_DIM, _N_EMBED), lambda i: (0, 0)),
        ],
        out_specs=[
            pl.BlockSpec((_BR, _COLS, _DIM), lambda i: (i, 0, 0)),
            pl.BlockSpec((_BLK,), lambda i: (i,)),
            pl.BlockSpec((1, 1), lambda i: (0, 0)),
            pl.BlockSpec((1, 1), lambda i: (0, 0)),
        ],
        out_shape=[
            jax.ShapeDtypeStruct((_ROWS, _COLS, _DIM), jnp.float32),
            jax.ShapeDtypeStruct((_TOKENS,), jnp.int32),
            jax.ShapeDtypeStruct((1, 1), jnp.float32),
            jax.ShapeDtypeStruct((1, 1), jnp.float32),
        ],
        scratch_shapes=[
            pltpu.VMEM((_N_EMBED,), jnp.float32),
            pltpu.SMEM((1, 1), jnp.float32),
        ],
    )(input, embed)
    return q, diff[0, 0], ind.reshape(_ROWS, _COLS), ppl[0, 0]


# speedup vs baseline: 1.1380x; 1.0309x over previous
import functools

import jax
import jax.numpy as jnp
from jax.experimental import pallas as pl
from jax.experimental.pallas import tpu as pltpu

_DIM = 64
_N_EMBED = 1024
_ROWS = 16
_COLS = 1024
_TOKENS = _ROWS * _COLS
_BR = 2
_BLK = _BR * _COLS
_NUM_CHUNKS = _ROWS // _BR


def _vq_body(x_ref, e_ref, q_ref, ind_ref, diff_ref, ppl_ref):
    e = e_ref[...]
    e_sq = jnp.sum(e * e, axis=0, keepdims=True)
    iota = jax.lax.broadcasted_iota(jnp.int32, (_BLK, _N_EMBED), 1)

    cnt = jnp.zeros((_N_EMBED,), dtype=jnp.float32)
    dsum = jnp.float32(0.0)
    for c in range(_NUM_CHUNKS):
        x = x_ref[c * _BR:(c + 1) * _BR].reshape(_BLK, _DIM)
        neg2_scores = jax.lax.dot_general(
            x * (-2.0), e, (((1,), (0,)), ((), ())),
            preferred_element_type=jnp.float32)
        x_sq = jnp.sum(x * x, axis=1, keepdims=True)
        dist = (x_sq + neg2_scores) + e_sq

        ind = jnp.argmin(dist, axis=1).astype(jnp.int32)
        onehot = (iota == ind[:, None]).astype(jnp.float32)
        q = jax.lax.dot_general(
            onehot, e, (((1,), (1,)), ((), ())),
            preferred_element_type=jnp.float32)

        q_ref[c * _BR:(c + 1) * _BR] = (x + (q - x)).reshape(_BR, _COLS, _DIM)
        ind_ref[c * _BLK:(c + 1) * _BLK] = ind

        ones = jnp.ones((1, _BLK), dtype=jnp.float32)
        cnt = cnt + jax.lax.dot_general(
            ones, onehot, (((1,), (0,)), ((), ())),
            preferred_element_type=jnp.float32)[0]
        dsum = dsum + jnp.sum((q - x) ** 2)

    diff_ref[...] = jnp.reshape(dsum / float(_TOKENS * _DIM), (1, 1))
    p = cnt / float(_TOKENS)
    ent = jnp.sum(p * jnp.log(jnp.clip(p, 1e-7, None)), keepdims=True)
    ppl_ref[...] = jnp.exp(-ent).reshape(1, 1)


@functools.partial(jax.jit, static_argnames=())
def kernel(input, embed):
    q, ind, diff, ppl = pl.pallas_call(
        _vq_body,
        out_shape=[
            jax.ShapeDtypeStruct((_ROWS, _COLS, _DIM), jnp.float32),
            jax.ShapeDtypeStruct((_TOKENS,), jnp.int32),
            jax.ShapeDtypeStruct((1, 1), jnp.float32),
            jax.ShapeDtypeStruct((1, 1), jnp.float32),
        ],
    )(input, embed)
    return q, diff[0, 0], ind.reshape(_ROWS, _COLS), ppl[0, 0]
